# baseline (speedup 1.0000x reference)
import functools

import jax
import jax.numpy as jnp
from jax import lax
from jax.experimental import pallas as pl
from jax.experimental.pallas import tpu as pltpu
from jax.experimental.pallas import tpu_sc as plsc

B = 16384
D = 32
V = 100001
NLINES = (V + 3) // 4

_info = plsc.get_sparse_core_info()
_NC = _info.num_cores
_NS = _info.num_subcores
_NW = _NC * _NS
_BPW = B // _NW

_mesh = plsc.VectorSubcoreMesh(core_axis_name="c", subcore_axis_name="s")


@functools.partial(
    pl.kernel,
    mesh=_mesh,
    out_type=jax.ShapeDtypeStruct((B, 4 * D), jnp.float32),
    scratch_types=[
        pltpu.VMEM((_BPW,), jnp.int32),
        pltpu.VMEM((_BPW,), jnp.int32),
        pltpu.VMEM((_BPW, 4 * D), jnp.float32),
        pltpu.SemaphoreType.DMA,
    ],
)
def _sc_gather(lines_hbm, idx_hbm, out_hbm, idx_v, j_v, rows_v, sem):
    wid = lax.axis_index("s") * _NC + lax.axis_index("c")
    base = wid * _BPW
    pltpu.sync_copy(idx_hbm.at[pl.ds(base, _BPW)], idx_v)
    for k in range(_BPW // 16):
        sl = pl.ds(k * 16, 16)
        j_v[sl] = lax.shift_right_logical(idx_v[sl], 2)
    pltpu.async_copy(lines_hbm.at[j_v], rows_v, sem).wait()
    pltpu.sync_copy(rows_v, out_hbm.at[pl.ds(base, _BPW)])


def _select_mlp_body(g_ref, uid_ref, w1, b1, w2, b2, o_ref):
    slot = uid_ref[...] & 3
    g = g_ref[...]
    x = jnp.where(
        slot < 2,
        jnp.where(slot == 0, g[:, 0:D], g[:, D:2 * D]),
        jnp.where(slot == 2, g[:, 2 * D:3 * D], g[:, 3 * D:4 * D]),
    )
    h = jnp.maximum(
        jnp.dot(x, w1[...], preferred_element_type=jnp.float32) + b1[...],
        0.0,
    )
    o_ref[...] = (
        jnp.dot(h, w2[...], preferred_element_type=jnp.float32) + b2[...]
    )


def _select_mlp(gathered, user_id, W1, b1, W2, b2):
    blk = 4096
    return pl.pallas_call(
        _select_mlp_body,
        grid=(B // blk,),
        in_specs=[
            pl.BlockSpec((blk, 4 * D), lambda i: (i, 0)),
            pl.BlockSpec((blk, 1), lambda i: (i, 0)),
            pl.BlockSpec(W1.shape, lambda i: (0, 0)),
            pl.BlockSpec((1, W1.shape[1]), lambda i: (0, 0)),
            pl.BlockSpec(W2.shape, lambda i: (0, 0)),
            pl.BlockSpec((1, W2.shape[1]), lambda i: (0, 0)),
        ],
        out_specs=pl.BlockSpec((blk, D), lambda i: (i, 0)),
        out_shape=jax.ShapeDtypeStruct((B, D), jnp.float32),
    )(
        gathered,
        user_id.reshape(B, 1),
        W1,
        b1.reshape(1, -1),
        W2,
        b2.reshape(1, -1),
    )


def kernel(user_id, table, W1, b1, W2, b2):
    uid = user_id.astype(jnp.int32)
    lines = jnp.pad(table, ((0, 4 * NLINES - V), (0, 0))).reshape(NLINES, 4 * D)
    gathered = _sc_gather(lines, uid)
    return _select_mlp(gathered, uid, W1, b1, W2, b2)


# speedup vs baseline: 1.4940x; 1.4940x over previous
import functools

import jax
import jax.numpy as jnp
from jax import lax
from jax.experimental import pallas as pl
from jax.experimental.pallas import tpu as pltpu
from jax.experimental.pallas import tpu_sc as plsc

B = 16384
D = 32
V = 100001

_info = plsc.get_sparse_core_info()
_NC = _info.num_cores
_NS = _info.num_subcores
_NW = _NC * _NS
_BPW = B // _NW

_mesh = plsc.VectorSubcoreMesh(core_axis_name="c", subcore_axis_name="s")


@functools.partial(
    pl.kernel,
    mesh=_mesh,
    out_type=jax.ShapeDtypeStruct((B, D), jnp.float32),
    scratch_types=[
        pltpu.VMEM((_BPW,), jnp.int32),
        pltpu.VMEM((_BPW, D), jnp.float32),
        pltpu.SemaphoreType.DMA,
    ],
    compiler_params=pltpu.CompilerParams(use_tc_tiling_on_sc=False),
)
def _sc_gather(table_hbm, idx_hbm, out_hbm, idx_v, rows_v, sem):
    wid = lax.axis_index("s") * _NC + lax.axis_index("c")
    base = wid * _BPW
    pltpu.sync_copy(idx_hbm.at[pl.ds(base, _BPW)], idx_v)
    pltpu.async_copy(table_hbm.at[idx_v], rows_v, sem).wait()
    pltpu.sync_copy(rows_v, out_hbm.at[pl.ds(base, _BPW)])


def _mlp_body(x_ref, w1_ref, b1_ref, w2_ref, b2_ref, o_ref):
    x = x_ref[...]
    h = jnp.maximum(
        jnp.dot(x, w1_ref[...], preferred_element_type=jnp.float32)
        + b1_ref[...],
        0.0,
    )
    o_ref[...] = (
        jnp.dot(h, w2_ref[...], preferred_element_type=jnp.float32)
        + b2_ref[...]
    )


def _packed_mlp(x_packed, W1p, b1p, W2p, b2p):
    blk = 1024
    n = x_packed.shape[0]
    return pl.pallas_call(
        _mlp_body,
        grid=(n // blk,),
        in_specs=[
            pl.BlockSpec((blk, 4 * D), lambda i: (i, 0)),
            pl.BlockSpec(W1p.shape, lambda i: (0, 0)),
            pl.BlockSpec((1, W1p.shape[1]), lambda i: (0, 0)),
            pl.BlockSpec(W2p.shape, lambda i: (0, 0)),
            pl.BlockSpec((1, W2p.shape[1]), lambda i: (0, 0)),
        ],
        out_specs=pl.BlockSpec((blk, 4 * D), lambda i: (i, 0)),
        out_shape=jax.ShapeDtypeStruct((n, 4 * D), jnp.float32),
    )(x_packed, W1p, b1p.reshape(1, -1), W2p, b2p.reshape(1, -1))


def kernel(user_id, table, W1, b1, W2, b2):
    uid = user_id.astype(jnp.int32)
    flat = lax.optimization_barrier(jnp.reshape(table, (V * D,)))
    t_lin = flat.reshape(V, D)
    gathered = _sc_gather(t_lin, uid)

    eye4 = jnp.eye(4, dtype=jnp.float32)
    W1p = jnp.kron(eye4, W1)
    W2p = jnp.kron(eye4, W2)
    b1p = jnp.tile(b1, 4)
    b2p = jnp.tile(b2, 4)

    x_packed = gathered.reshape(B // 4, 4 * D)
    out_packed = _packed_mlp(x_packed, W1p, b1p, W2p, b2p)
    return out_packed.reshape(B, D)
